# baseline (speedup 1.0000x reference)
import functools

import jax
import jax.numpy as jnp
from jax import lax
from jax.experimental import pallas as pl
from jax.experimental.pallas import tpu as pltpu
from jax.experimental.pallas import tpu_sc as plsc

_ROWS, _COLS = 16384, 4096
_BLOCK_ROWS = 512
_STRIDE = 64


def _mask_copy_kernel(x_ref, o_ref):
    lane = jax.lax.broadcasted_iota(jnp.int32, (_BLOCK_ROWS, _COLS), 1)
    keep = (lane % _STRIDE) != 0
    o_ref[...] = jnp.where(keep, x_ref[...], 0.0)


def _tc_kernel(x):
    grid = (_ROWS // _BLOCK_ROWS,)
    return pl.pallas_call(
        _mask_copy_kernel,
        grid=grid,
        in_specs=[pl.BlockSpec((_BLOCK_ROWS, _COLS), lambda i: (i, 0))],
        out_specs=pl.BlockSpec((_BLOCK_ROWS, _COLS), lambda i: (i, 0)),
        out_shape=jax.ShapeDtypeStruct((_ROWS, _COLS), x.dtype),
        compiler_params=pltpu.CompilerParams(
            dimension_semantics=("parallel",),
        ),
    )(x)


_NW = 32
_NWORDS = _ROWS * _COLS
_WORDS_PER_W = _NWORDS // _NW
_CHUNK = 32768
_NCHUNK = _WORDS_PER_W // _CHUNK


@functools.partial(
    pl.kernel,
    out_type=jax.ShapeDtypeStruct((_NWORDS,), jnp.float32),
    mesh=plsc.VectorSubcoreMesh(core_axis_name="c", subcore_axis_name="s"),
    scratch_types=[pltpu.VMEM((_CHUNK,), jnp.float32)],
    compiler_params=pltpu.CompilerParams(needs_layout_passes=False),
)
def _sc_zero_mask(x_hbm, out_hbm, buf):
    wid = lax.axis_index("s") * 2 + lax.axis_index("c")
    idx16 = lax.iota(jnp.int32, 16) * _STRIDE
    zeros = jnp.zeros((16,), jnp.float32)

    def body(g, carry):
        base = wid * _WORDS_PER_W + g * _CHUNK
        pltpu.sync_copy(x_hbm.at[pl.ds(base, _CHUNK)], buf)
        for k in range(_CHUNK // (16 * _STRIDE)):
            plsc.store_scatter(buf, [idx16 + k * (16 * _STRIDE)], zeros)
        pltpu.sync_copy(buf, out_hbm.at[pl.ds(base, _CHUNK)])
        return carry

    lax.fori_loop(0, _NCHUNK, body, 0)


def _sc_kernel(x):
    return _sc_zero_mask(x.reshape(-1)).reshape(_ROWS, _COLS)


def kernel(x):
    return _sc_kernel(x)


# speedup vs baseline: 1.2095x; 1.2095x over previous
import functools

import jax
import jax.numpy as jnp
from jax import lax
from jax.experimental import pallas as pl
from jax.experimental.pallas import tpu as pltpu
from jax.experimental.pallas import tpu_sc as plsc

_ROWS, _COLS = 16384, 4096
_BLOCK_ROWS = 512
_STRIDE = 64


def _mask_copy_kernel(x_ref, o_ref):
    lane = jax.lax.broadcasted_iota(jnp.int32, (_BLOCK_ROWS, _COLS), 1)
    keep = (lane % _STRIDE) != 0
    o_ref[...] = jnp.where(keep, x_ref[...], 0.0)


def _tc_kernel(x):
    grid = (_ROWS // _BLOCK_ROWS,)
    return pl.pallas_call(
        _mask_copy_kernel,
        grid=grid,
        in_specs=[pl.BlockSpec((_BLOCK_ROWS, _COLS), lambda i: (i, 0))],
        out_specs=pl.BlockSpec((_BLOCK_ROWS, _COLS), lambda i: (i, 0)),
        out_shape=jax.ShapeDtypeStruct((_ROWS, _COLS), x.dtype),
        compiler_params=pltpu.CompilerParams(
            dimension_semantics=("parallel",),
        ),
    )(x)


_NW = 32
_NWORDS = _ROWS * _COLS
_WORDS_PER_W = _NWORDS // _NW
_CHUNK = 32768
_NCHUNK = _WORDS_PER_W // _CHUNK


@functools.partial(
    pl.kernel,
    out_type=jax.ShapeDtypeStruct((_NWORDS,), jnp.float32),
    mesh=plsc.VectorSubcoreMesh(core_axis_name="c", subcore_axis_name="s"),
    scratch_types=[pltpu.VMEM((_CHUNK,), jnp.float32)],
    compiler_params=pltpu.CompilerParams(needs_layout_passes=False),
)
def _sc_zero_mask(x_hbm, out_hbm, buf):
    wid = lax.axis_index("s") * 2 + lax.axis_index("c")
    idx16 = lax.iota(jnp.int32, 16) * _STRIDE
    zeros = jnp.zeros((16,), jnp.float32)

    def body(g, carry):
        base = wid * _WORDS_PER_W + g * _CHUNK
        pltpu.sync_copy(x_hbm.at[pl.ds(base, _CHUNK)], buf)
        for k in range(_CHUNK // (16 * _STRIDE)):
            plsc.store_scatter(buf, [idx16 + k * (16 * _STRIDE)], zeros)
        pltpu.sync_copy(buf, out_hbm.at[pl.ds(base, _CHUNK)])
        return carry

    lax.fori_loop(0, _NCHUNK, body, 0)


def _sc_kernel(x):
    return _sc_zero_mask(x.reshape(-1)).reshape(_ROWS, _COLS)


_SC_ROWS = 3072
_TC_ROWS = _ROWS - _SC_ROWS
_SC_WORDS = _SC_ROWS * _COLS
_SC_WORDS_PER_W = _SC_WORDS // _NW
_SC_NCHUNK = _SC_WORDS_PER_W // _CHUNK


def _tc_part(x):
    grid = (_TC_ROWS // _BLOCK_ROWS,)
    return pl.pallas_call(
        _mask_copy_kernel,
        grid=grid,
        in_specs=[pl.BlockSpec((_BLOCK_ROWS, _COLS), lambda i: (i, 0))],
        out_specs=pl.BlockSpec((_BLOCK_ROWS, _COLS), lambda i: (i, 0)),
        out_shape=jax.ShapeDtypeStruct((_TC_ROWS, _COLS), x.dtype),
        compiler_params=pltpu.CompilerParams(
            dimension_semantics=("parallel",),
        ),
    )(x)


@functools.partial(
    pl.kernel,
    out_type=jax.ShapeDtypeStruct((_SC_WORDS,), jnp.float32),
    mesh=plsc.VectorSubcoreMesh(core_axis_name="c", subcore_axis_name="s"),
    scratch_types=[pltpu.VMEM((_CHUNK,), jnp.float32)],
    compiler_params=pltpu.CompilerParams(needs_layout_passes=False),
)
def _sc_zero_mask_part(x_hbm, out_hbm, buf):
    wid = lax.axis_index("s") * 2 + lax.axis_index("c")
    idx16 = lax.iota(jnp.int32, 16) * _STRIDE
    zeros = jnp.zeros((16,), jnp.float32)

    def body(g, carry):
        base = wid * _SC_WORDS_PER_W + g * _CHUNK
        pltpu.sync_copy(x_hbm.at[pl.ds(_TC_ROWS * _COLS + base, _CHUNK)], buf)
        for k in range(_CHUNK // (16 * _STRIDE)):
            plsc.store_scatter(buf, [idx16 + k * (16 * _STRIDE)], zeros)
        pltpu.sync_copy(buf, out_hbm.at[pl.ds(base, _CHUNK)])
        return carry

    lax.fori_loop(0, _SC_NCHUNK, body, 0)


def _hybrid_kernel(x):
    top = _tc_part(x)
    bot = _sc_zero_mask_part(x.reshape(-1)).reshape(_SC_ROWS, _COLS)
    return jnp.concatenate([top, bot], axis=0)


def kernel(x):
    return _hybrid_kernel(x)


# speedup vs baseline: 4.3051x; 3.5595x over previous
import jax
import jax.numpy as jnp
from jax.experimental import pallas as pl

_ROWS, _COLS = 16384, 4096
_BLOCK_ROWS = 512
_STRIDE = 64


def _mask_copy_kernel(x_ref, o_ref):
    lane = jax.lax.broadcasted_iota(jnp.int32, (_BLOCK_ROWS, _COLS), 1)
    keep = (lane % _STRIDE) != 0
    o_ref[...] = jnp.where(keep, x_ref[...], 0.0)


def kernel(x):
    grid = (_ROWS // _BLOCK_ROWS,)
    return pl.pallas_call(
        _mask_copy_kernel,
        grid=grid,
        in_specs=[pl.BlockSpec((_BLOCK_ROWS, _COLS), lambda i: (i, 0))],
        out_specs=pl.BlockSpec((_BLOCK_ROWS, _COLS), lambda i: (i, 0)),
        out_shape=jax.ShapeDtypeStruct((_ROWS, _COLS), x.dtype),
    )(x)
